# baseline (speedup 1.0000x reference)
import functools

import jax
import jax.numpy as jnp
from jax import lax
from jax.experimental import pallas as pl
from jax.experimental.pallas import tpu as pltpu
from jax.experimental.pallas import tpu_sc as plsc

N_NODES = 10000
D_FEAT = 256
DQ = 64
NQ = 4
NR = N_NODES // 2
CHUNK = 128
NSUB = 16
NCORE = 2
EPC = 10240
NCHUNK = EPC // CHUNK
EPAD = NSUB * EPC
RPS = 312
TAIL = NR - NSUB * RPS


def _spmm_body(src_hbm, cols_hbm, cpar_hbm, rows_hbm, rpar_hbm, vals_hbm,
               out_hbm, colbuf, rowidxbuf, valbuf, cparbuf, rparbuf,
               gbuf, fbuf, zerobuf, xstage, accum, sem0):
    c = lax.axis_index("c")
    s = lax.axis_index("s")
    base_r = s * RPS

    zero16 = jnp.zeros((16,), jnp.float32)

    def zb_body(i, carry):
        for j in range(8):
            zerobuf[i, pl.ds(j * 16, 16)] = zero16
        return carry

    lax.fori_loop(0, 128, zb_body, 0)

    ebase = s * EPC
    dn = lax.GatherDimensionNumbers(offset_dims=(),
                                    collapsed_slice_dims=(0,),
                                    start_index_map=(0,))
    lane_idx = [jnp.full((16, 1), i, jnp.int32) for i in range(16)]

    def bcast(grp, i):
        return lax.gather(grp, lane_idx[i], dn, slice_sizes=(1,),
                          mode=lax.GatherScatterMode.PROMISE_IN_BOUNDS)

    def process(k):
        def grp_body(g, gcarry):
            vgrp = valbuf[pl.ds(g * 16, 16)]
            cgrp = cparbuf[pl.ds(g * 16, 16)]
            rgrp = rparbuf[pl.ds(g * 16, 16)]
            one = jnp.ones((16,), jnp.float32)
            for i in range(16):
                vb = bcast(vgrp, i)
                cp = bcast(cgrp, i)
                rp = bcast(rgrp, i)
                cpv = cp * vb
                cpnv = (one - cp) * vb
                rpn = one - rp
                e = g * 16 + i
                for j in range(4):
                    u = gbuf[e, pl.ds(j * 16, 16)]
                    w = gbuf[e, pl.ds(64 + j * 16, 16)]
                    prod = u * cpnv + w * cpv
                    fbuf[e, pl.ds(j * 16, 16)] = prod * rpn
                    fbuf[e, pl.ds(64 + j * 16, 16)] = prod * rp
            return gcarry

        lax.fori_loop(0, CHUNK // 16, grp_body, 0)
        pltpu.sync_copy(fbuf, accum.at[rowidxbuf], add=True)

    for q in range(2):
        base = (2 * c + q) * NR

        def stage_block(src_row0, dst_row0, nrows):
            pltpu.sync_copy(src_hbm.at[pl.ds(src_row0, nrows)],
                            gbuf.at[pl.ds(0, nrows)])
            pltpu.sync_copy(gbuf.at[pl.ds(0, nrows)],
                            xstage.at[pl.ds(dst_row0, nrows)])

        for t, nrows in enumerate((128, 128, RPS - 256)):
            stage_block(base + base_r + t * 128, base_r + t * 128, nrows)
            pltpu.sync_copy(zerobuf.at[pl.ds(0, nrows)],
                            accum.at[pl.ds(base_r + t * 128, nrows)])

        @pl.when(s == NSUB - 1)
        def _stage_tail():
            stage_block(base + NSUB * RPS, NSUB * RPS, TAIL)
            pltpu.sync_copy(zerobuf.at[pl.ds(0, TAIL)],
                            accum.at[pl.ds(NSUB * RPS, TAIL)])

        plsc.subcore_barrier()

        def chunk_body(k, carry):
            off = ebase + k * CHUNK
            pltpu.sync_copy(cols_hbm.at[pl.ds(off, CHUNK)], colbuf)
            pltpu.sync_copy(rows_hbm.at[pl.ds(off, CHUNK)], rowidxbuf)
            pltpu.sync_copy(vals_hbm.at[pl.ds(off, CHUNK)], valbuf)
            pltpu.sync_copy(cpar_hbm.at[pl.ds(off, CHUNK)], cparbuf)
            pltpu.sync_copy(rpar_hbm.at[pl.ds(off, CHUNK)], rparbuf)
            pltpu.async_copy(xstage.at[colbuf], gbuf, sem0)
            pltpu.make_async_copy(xstage.at[colbuf], gbuf, sem0).wait()
            process(k)
            return carry

        lax.fori_loop(0, NCHUNK, chunk_body, 0)
        plsc.subcore_barrier()

        pltpu.sync_copy(accum.at[pl.ds(base_r, RPS)],
                        out_hbm.at[pl.ds(base + base_r, RPS)])

        @pl.when(s == NSUB - 1)
        def _write_tail():
            pltpu.sync_copy(accum.at[pl.ds(NSUB * RPS, TAIL)],
                            out_hbm.at[pl.ds(base + NSUB * RPS, TAIL)])


@functools.lru_cache(maxsize=None)
def _get_spmm_kernel():
    return pl.kernel(
        _spmm_body,
        out_type=jax.ShapeDtypeStruct((NQ * NR, 2 * DQ), jnp.float32),
        mesh=plsc.VectorSubcoreMesh(core_axis_name="c", subcore_axis_name="s"),
        scratch_types=[
            pltpu.VMEM((CHUNK,), jnp.int32),
            pltpu.VMEM((CHUNK,), jnp.int32),
            pltpu.VMEM((CHUNK,), jnp.float32),
            pltpu.VMEM((CHUNK,), jnp.float32),
            pltpu.VMEM((CHUNK,), jnp.float32),
            pltpu.VMEM((CHUNK, 2 * DQ), jnp.float32),
            pltpu.VMEM((CHUNK, 2 * DQ), jnp.float32),
            pltpu.VMEM((128, 2 * DQ), jnp.float32),
            pltpu.VMEM_SHARED((NR, 2 * DQ), jnp.float32),
            pltpu.VMEM_SHARED((NR, 2 * DQ), jnp.float32),
            pltpu.SemaphoreType.DMA,
        ],
    )


def _linear_body(x_ref, t1_ref, s2_ref, w_ref, b_ref, o_ref):
    xb = x_ref[...]
    wa = w_ref[:, 0:256]
    w1 = w_ref[:, 256:512]
    wc = w_ref[:, 512:768]
    dn = (((1,), (1,)), ((), ()))
    acc = lax.dot_general(xb, wa, dn, preferred_element_type=jnp.float32)
    for f in range(NQ):
        acc = acc + lax.dot_general(
            t1_ref[f], w1[:, f * DQ:(f + 1) * DQ], dn,
            preferred_element_type=jnp.float32)
        acc = acc + lax.dot_general(
            s2_ref[f], wc[:, f * DQ:(f + 1) * DQ], dn,
            preferred_element_type=jnp.float32)
    o_ref[...] = acc + b_ref[...]


def _linear(x, t1q, s2q, Wcat, b):
    R = 1000
    grid = (N_NODES // R,)
    return pl.pallas_call(
        _linear_body,
        grid=grid,
        in_specs=[
            pl.BlockSpec((R, D_FEAT), lambda i: (i, 0)),
            pl.BlockSpec((NQ, R, DQ), lambda i: (0, i, 0)),
            pl.BlockSpec((NQ, R, DQ), lambda i: (0, i, 0)),
            pl.BlockSpec((D_FEAT, 3 * D_FEAT), lambda i: (0, 0)),
            pl.BlockSpec((1, D_FEAT), lambda i: (0, 0)),
        ],
        out_specs=pl.BlockSpec((R, D_FEAT), lambda i: (i, 0)),
        out_shape=jax.ShapeDtypeStruct((N_NODES, D_FEAT), jnp.float32),
    )(x, t1q, s2q, Wcat, b.reshape(1, D_FEAT))


def kernel(L_indices, L_values, x, W, b):
    rows = L_indices[0].astype(jnp.int32)
    cols = L_indices[1].astype(jnp.int32)
    n_edges = rows.shape[0]
    pad = EPAD - n_edges
    rows_p = jnp.pad(rows, (0, pad))
    cols_p = jnp.pad(cols, (0, pad))
    vals_p = jnp.pad(L_values, (0, pad))
    cols2 = cols_p // 2
    cpar = (cols_p % 2).astype(jnp.float32)
    rows2 = rows_p // 2
    rpar = (rows_p % 2).astype(jnp.float32)

    xq = jnp.concatenate(
        [x[:, f * DQ:(f + 1) * DQ].reshape(NR, 2 * DQ) for f in range(NQ)],
        axis=0)

    W0 = W[:, 0:D_FEAT]
    W1 = W[:, D_FEAT:2 * D_FEAT]
    W2 = W[:, 2 * D_FEAT:]
    Wcat = jnp.concatenate([W0 - W2, W1, 2.0 * W2], axis=1)

    spmm = _get_spmm_kernel()
    t1s = spmm(xq, cols2, cpar, rows2, rpar, vals_p)
    s2s = spmm(t1s, cols2, cpar, rows2, rpar, vals_p)
    t1q = t1s.reshape(NQ, N_NODES, DQ)
    s2q = s2s.reshape(NQ, N_NODES, DQ)
    return _linear(x, t1q, s2q, Wcat, b)


# speedup vs baseline: 2.2206x; 2.2206x over previous
import functools

import jax
import jax.numpy as jnp
from jax import lax
from jax.experimental import pallas as pl
from jax.experimental.pallas import tpu as pltpu
from jax.experimental.pallas import tpu_sc as plsc

N_NODES = 10000
D_FEAT = 256
DH = 128
CHUNK = 128
NSUB = 16
NCORE = 2
EPC = 10240
NCHUNK = EPC // CHUNK
NPAIR = NCHUNK // 2
EPAD = NSUB * EPC
RPS = 624


def _spmm_body(xs_hbm, cols2_hbm, rows_hbm, vals_hbm, out_hbm,
               colbuf0, colbuf1, rowidxbuf, valbuf, rowbuf0, rowbuf1,
               zerobuf, accum, sem0, sem1):
    c = lax.axis_index("c")
    s = lax.axis_index("s")

    zero16 = jnp.zeros((16,), jnp.float32)

    def zb_body(i, carry):
        for j in range(8):
            zerobuf[i, pl.ds(j * 16, 16)] = zero16
        return carry

    lax.fori_loop(0, 128, zb_body, 0)
    base_r = s * RPS
    for t in range(4):
        pltpu.sync_copy(zerobuf.at[:], accum.at[pl.ds(base_r + t * 128, 128)])
    pltpu.sync_copy(zerobuf.at[pl.ds(0, RPS - 512)],
                    accum.at[pl.ds(base_r + 512, RPS - 512)])

    tail = N_NODES - NSUB * RPS

    @pl.when(s == NSUB - 1)
    def _zero_tail():
        pltpu.sync_copy(zerobuf.at[pl.ds(0, tail)],
                        accum.at[pl.ds(NSUB * RPS, tail)])

    plsc.subcore_barrier()

    ebase = s * EPC
    dn = lax.GatherDimensionNumbers(offset_dims=(),
                                    collapsed_slice_dims=(0,),
                                    start_index_map=(0,))
    lane_idx = [jnp.full((16, 1), i, jnp.int32) for i in range(16)]

    def start_gather(k, colbuf, rowbuf, sem):
        off = ebase + k * CHUNK
        pltpu.sync_copy(cols2_hbm.at[c, pl.ds(off, CHUNK)], colbuf)
        pltpu.async_copy(xs_hbm.at[colbuf], rowbuf, sem)

    def wait_gather(colbuf, rowbuf, sem):
        pltpu.make_async_copy(xs_hbm.at[colbuf], rowbuf, sem).wait()

    def process(k, rowbuf):
        off = ebase + k * CHUNK
        pltpu.sync_copy(rows_hbm.at[pl.ds(off, CHUNK)], rowidxbuf)
        pltpu.sync_copy(vals_hbm.at[pl.ds(off, CHUNK)], valbuf)

        def grp_body(g, gcarry):
            grp = valbuf[pl.ds(g * 16, 16)]
            for i in range(16):
                vb = lax.gather(grp, lane_idx[i], dn, slice_sizes=(1,),
                                mode=lax.GatherScatterMode.PROMISE_IN_BOUNDS)
                e = g * 16 + i
                for j in range(8):
                    rowbuf[e, pl.ds(j * 16, 16)] = (
                        rowbuf[e, pl.ds(j * 16, 16)] * vb)
            return gcarry

        lax.fori_loop(0, CHUNK // 16, grp_body, 0)
        pltpu.sync_copy(rowbuf, accum.at[rowidxbuf], add=True)

    start_gather(0, colbuf0, rowbuf0, sem0)

    def pair_body(p, carry):
        start_gather(2 * p + 1, colbuf1, rowbuf1, sem1)
        wait_gather(colbuf0, rowbuf0, sem0)
        process(2 * p, rowbuf0)

        @pl.when(p < NPAIR - 1)
        def _prefetch_even():
            start_gather(2 * p + 2, colbuf0, rowbuf0, sem0)

        wait_gather(colbuf1, rowbuf1, sem1)
        process(2 * p + 1, rowbuf1)
        return carry

    lax.fori_loop(0, NPAIR, pair_body, 0)
    plsc.subcore_barrier()

    out_base = c * N_NODES + s * RPS
    pltpu.sync_copy(accum.at[pl.ds(base_r, RPS)],
                    out_hbm.at[pl.ds(out_base, RPS)])

    @pl.when(s == NSUB - 1)
    def _write_tail():
        pltpu.sync_copy(accum.at[pl.ds(NSUB * RPS, tail)],
                        out_hbm.at[pl.ds(c * N_NODES + NSUB * RPS, tail)])


@functools.lru_cache(maxsize=None)
def _get_spmm_kernel():
    return pl.kernel(
        _spmm_body,
        out_type=jax.ShapeDtypeStruct((NCORE * N_NODES, DH), jnp.float32),
        mesh=plsc.VectorSubcoreMesh(core_axis_name="c", subcore_axis_name="s"),
        scratch_types=[
            pltpu.VMEM((CHUNK,), jnp.int32),
            pltpu.VMEM((CHUNK,), jnp.int32),
            pltpu.VMEM((CHUNK,), jnp.int32),
            pltpu.VMEM((CHUNK,), jnp.float32),
            pltpu.VMEM((CHUNK, DH), jnp.float32),
            pltpu.VMEM((CHUNK, DH), jnp.float32),
            pltpu.VMEM((128, DH), jnp.float32),
            pltpu.VMEM_SHARED((N_NODES, DH), jnp.float32),
            pltpu.SemaphoreType.DMA,
            pltpu.SemaphoreType.DMA,
        ],
    )


def _linear_body(x_ref, t1_ref, s2_ref, w_ref, b_ref, o_ref):
    xb = x_ref[...]
    w0 = w_ref[:, 0:256]
    w1 = w_ref[:, 256:512]
    w2 = w_ref[:, 512:768]
    t1a = t1_ref[0]
    t1b = t1_ref[1]
    t2a = 2.0 * s2_ref[0] - xb[:, :DH]
    t2b = 2.0 * s2_ref[1] - xb[:, DH:]
    dn = (((1,), (1,)), ((), ()))
    acc = lax.dot_general(xb, w0, dn, preferred_element_type=jnp.float32)
    acc = acc + lax.dot_general(t1a, w1[:, :DH], dn,
                                preferred_element_type=jnp.float32)
    acc = acc + lax.dot_general(t1b, w1[:, DH:], dn,
                                preferred_element_type=jnp.float32)
    acc = acc + lax.dot_general(t2a, w2[:, :DH], dn,
                                preferred_element_type=jnp.float32)
    acc = acc + lax.dot_general(t2b, w2[:, DH:], dn,
                                preferred_element_type=jnp.float32)
    o_ref[...] = acc + b_ref[...]


def _linear(x, t1r, s2r, W, b):
    R = 1000
    grid = (N_NODES // R,)
    return pl.pallas_call(
        _linear_body,
        grid=grid,
        in_specs=[
            pl.BlockSpec((R, D_FEAT), lambda i: (i, 0)),
            pl.BlockSpec((NCORE, R, DH), lambda i: (0, i, 0)),
            pl.BlockSpec((NCORE, R, DH), lambda i: (0, i, 0)),
            pl.BlockSpec((D_FEAT, 3 * D_FEAT), lambda i: (0, 0)),
            pl.BlockSpec((1, D_FEAT), lambda i: (0, 0)),
        ],
        out_specs=pl.BlockSpec((R, D_FEAT), lambda i: (i, 0)),
        out_shape=jax.ShapeDtypeStruct((N_NODES, D_FEAT), jnp.float32),
    )(x, t1r, s2r, W, b.reshape(1, D_FEAT))


def kernel(L_indices, L_values, x, W, b):
    rows = L_indices[0].astype(jnp.int32)
    cols = L_indices[1].astype(jnp.int32)
    n_edges = rows.shape[0]
    pad = EPAD - n_edges
    rows_p = jnp.pad(rows, (0, pad))
    cols_p = jnp.pad(cols, (0, pad))
    vals_p = jnp.pad(L_values, (0, pad))
    cols2 = jnp.stack([cols_p, cols_p + N_NODES])
    xs = jnp.concatenate([x[:, :DH], x[:, DH:]], axis=0)
    spmm = _get_spmm_kernel()
    t1s = spmm(xs, cols2, rows_p, vals_p)
    s2s = spmm(t1s, cols2, rows_p, vals_p)
    t1r = t1s.reshape(NCORE, N_NODES, DH)
    s2r = s2s.reshape(NCORE, N_NODES, DH)
    return _linear(x, t1r, s2r, W, b)
